# baseline (speedup 1.0000x reference)
import functools

import jax
import jax.numpy as jnp
from jax.experimental import pallas as pl
from jax.experimental.pallas import tpu as pltpu

N = 10000
BLK_M = 200
P = N // BLK_M


def _fused_body(adj_ref, x_ref, w1_ref, b1_ref, w2_ref, b2_ref, o_ref,
                sup_ref, z_ref):
    i = pl.program_id(0)

    @pl.when(i == 0)
    def _prologue():
        sup_ref[...] = jnp.dot(x_ref[...], w1_ref[...],
                               preferred_element_type=jnp.float32)

    @pl.when(i < P)
    def _phase1():
        acc = jnp.dot(adj_ref[...], sup_ref[...],
                      preferred_element_type=jnp.float32)
        pre = acc + b1_ref[...]
        h = jnp.where(pre > 0, pre, jnp.exp(jnp.minimum(pre, 0.0)) - 1.0)
        z_ref[pl.ds(i * BLK_M, BLK_M), :] = jnp.dot(
            h, w2_ref[...], preferred_element_type=jnp.float32)

    @pl.when(i >= P)
    def _phase2():
        acc = jnp.dot(adj_ref[...], z_ref[...],
                      preferred_element_type=jnp.float32)
        o_ref[...] = acc + b2_ref[...]


@functools.partial(jax.jit, static_argnames=())
def kernel(x, adj, W1, b1, W2, b2):
    n, f_in = x.shape
    hid = W1.shape[1]
    c = W2.shape[1]
    b1r = b1.reshape(1, hid)
    b2r = b2.reshape(1, c)

    out = pl.pallas_call(
        _fused_body,
        grid=(2 * P,),
        in_specs=[
            pl.BlockSpec((BLK_M, n), lambda i: (i % P, 0)),
            pl.BlockSpec((n, f_in), lambda i: (0, 0)),
            pl.BlockSpec((f_in, hid), lambda i: (0, 0)),
            pl.BlockSpec((1, hid), lambda i: (0, 0)),
            pl.BlockSpec((hid, c), lambda i: (0, 0)),
            pl.BlockSpec((1, c), lambda i: (0, 0)),
        ],
        out_specs=pl.BlockSpec((BLK_M, c), lambda i: (jnp.maximum(i - P, 0), 0)),
        out_shape=jax.ShapeDtypeStruct((n, c), jnp.float32),
        scratch_shapes=[
            pltpu.VMEM((n, hid), jnp.float32),
            pltpu.VMEM((n, c), jnp.float32),
        ],
        compiler_params=pltpu.CompilerParams(
            dimension_semantics=("arbitrary",)),
    )(adj, x, W1, b1r, W2, b2r)

    return out


# speedup vs baseline: 1.0313x; 1.0313x over previous
import functools

import jax
import jax.numpy as jnp
from jax.experimental import pallas as pl
from jax.experimental.pallas import tpu as pltpu

N = 10000
BLK_M = 400
P = N // BLK_M


def _fused_body(adj_ref, x_ref, w1_ref, b1_ref, w2_ref, b2_ref, o_ref,
                sup_ref, z_ref):
    i = pl.program_id(0)

    @pl.when(i == 0)
    def _prologue():
        sup_ref[...] = jnp.dot(x_ref[...], w1_ref[...],
                               preferred_element_type=jnp.float32)

    @pl.when(i < P)
    def _phase1():
        acc = jnp.dot(adj_ref[...], sup_ref[...],
                      preferred_element_type=jnp.float32)
        pre = acc + b1_ref[...]
        h = jnp.where(pre > 0, pre, jnp.exp(jnp.minimum(pre, 0.0)) - 1.0)
        z_ref[pl.ds(i * BLK_M, BLK_M), :] = jnp.dot(
            h, w2_ref[...], preferred_element_type=jnp.float32)

    @pl.when(i >= P)
    def _phase2():
        acc = jnp.dot(adj_ref[...], z_ref[...],
                      preferred_element_type=jnp.float32)
        o_ref[...] = acc + b2_ref[...]


@functools.partial(jax.jit, static_argnames=())
def kernel(x, adj, W1, b1, W2, b2):
    n, f_in = x.shape
    hid = W1.shape[1]
    c = W2.shape[1]
    b1r = b1.reshape(1, hid)
    b2r = b2.reshape(1, c)

    out = pl.pallas_call(
        _fused_body,
        grid=(2 * P,),
        in_specs=[
            pl.BlockSpec((BLK_M, n), lambda i: (i % P, 0)),
            pl.BlockSpec((n, f_in), lambda i: (0, 0)),
            pl.BlockSpec((f_in, hid), lambda i: (0, 0)),
            pl.BlockSpec((1, hid), lambda i: (0, 0)),
            pl.BlockSpec((hid, c), lambda i: (0, 0)),
            pl.BlockSpec((1, c), lambda i: (0, 0)),
        ],
        out_specs=pl.BlockSpec((BLK_M, c), lambda i: (jnp.maximum(i - P, 0), 0)),
        out_shape=jax.ShapeDtypeStruct((n, c), jnp.float32),
        scratch_shapes=[
            pltpu.VMEM((n, hid), jnp.float32),
            pltpu.VMEM((n, c), jnp.float32),
        ],
        compiler_params=pltpu.CompilerParams(
            dimension_semantics=("arbitrary",)),
    )(adj, x, W1, b1r, W2, b2r)

    return out
